# baseline (speedup 1.0000x reference)
import jax
import jax.numpy as jnp
from jax.experimental import pallas as pl
from jax.experimental.pallas import tpu as pltpu

N_ROWS = 32768
D = 768
NA = 18
NAPAD = 24
KSEL = 50
KPAD = 64
NBLK = 8
BLK = N_ROWS // NBLK
NSTR = 4
SW = BLK // NSTR


def _body(x_hbm, x_ref, w_ref, scores_out, spans_out, cls_out, emb_out,
          max_scr, key_scr, accv_scr, acck_scr, v_smem, k_smem, sem):
    g = pl.program_id(0)
    xb = x_ref[...].astype(jnp.bfloat16)
    st = jax.lax.dot_general(w_ref[...], xb, (((1,), (1,)), ((), ())),
                             preferred_element_type=jnp.float32)
    row = jax.lax.broadcasted_iota(jnp.int32, (NAPAD, 1), 0)
    m = jnp.max(st, axis=0)
    cls = jnp.min(jnp.where(st == m[None, :], row, NAPAD),
                  axis=0).astype(jnp.int32)
    col = jax.lax.iota(jnp.int32, BLK)
    max_scr[g, :] = m
    key_scr[g, :] = (g * BLK + col) * 32 + cls

    @pl.when(g == NBLK - 1)
    def _():
        lane = jax.lax.broadcasted_iota(jnp.int32, (1, 128), 1)
        for s in range(NSTR):
            a = max_scr[:, s * SW:(s + 1) * SW]
            keys = key_scr[:, s * SW:(s + 1) * SW]
            accv = jnp.zeros((1, 128), jnp.float32)
            acck = jnp.zeros((1, 128), jnp.int32)
            for i in range(KSEL):
                mm = jnp.max(a)
                cand = jnp.where(a == mm, keys, jnp.int32(2**30))
                j = jnp.min(cand)
                oh = lane == i
                accv = jnp.where(oh, mm, accv)
                acck = jnp.where(oh, j, acck)
                a = jnp.where(cand == j, -jnp.inf, a)
            accv_scr[s, :] = accv[0]
            acck_scr[s, :] = acck[0]
        cp = pltpu.make_async_copy(accv_scr, v_smem, sem)
        cp.start()
        cp.wait()
        cp = pltpu.make_async_copy(acck_scr, k_smem, sem)
        cp.start()
        cp.wait()
        ptrs = [jnp.int32(0)] * NSTR
        for i in range(KSEL):
            bv = v_smem[0, ptrs[0]]
            bk = k_smem[0, ptrs[0]]
            bs = jnp.int32(0)
            for s in range(1, NSTR):
                hv = v_smem[s, ptrs[s]]
                hk = k_smem[s, ptrs[s]]
                take = (hv > bv) | ((hv == bv) & (hk < bk))
                bv = jnp.where(take, hv, bv)
                bk = jnp.where(take, hk, bk)
                bs = jnp.where(take, jnp.int32(s), bs)
            scores_out[i] = bv
            spans_out[i] = jax.lax.shift_right_logical(bk, 5)
            cls_out[i] = jax.lax.bitwise_and(bk, 31)
            ptrs = [ptrs[s] + (bs == s).astype(jnp.int32)
                    for s in range(NSTR)]
        for i in range(KSEL):
            pltpu.make_async_copy(
                x_hbm.at[pl.ds(spans_out[i], 1), :],
                emb_out.at[pl.ds(i, 1), :], sem).start()
        for i in range(KSEL):
            pltpu.make_async_copy(
                x_hbm.at[pl.ds(0, 1), :],
                emb_out.at[pl.ds(i, 1), :], sem).wait()


def kernel(embs, entity_anchor, k):
    del k
    w_pad = jnp.concatenate(
        [entity_anchor,
         jnp.broadcast_to(entity_anchor[:1], (NAPAD - NA, D))],
        axis=0).astype(jnp.bfloat16)
    scores, spans, cls, emb = pl.pallas_call(
        _body,
        grid=(NBLK,),
        in_specs=[
            pl.BlockSpec(memory_space=pl.ANY),
            pl.BlockSpec((BLK, D), lambda g: (g, 0)),
            pl.BlockSpec((NAPAD, D), lambda g: (0, 0)),
        ],
        out_specs=[
            pl.BlockSpec(memory_space=pltpu.SMEM),
            pl.BlockSpec(memory_space=pltpu.SMEM),
            pl.BlockSpec(memory_space=pltpu.SMEM),
            pl.BlockSpec((KPAD, D), lambda g: (0, 0)),
        ],
        out_shape=[
            jax.ShapeDtypeStruct((128,), jnp.float32),
            jax.ShapeDtypeStruct((128,), jnp.int32),
            jax.ShapeDtypeStruct((128,), jnp.int32),
            jax.ShapeDtypeStruct((KPAD, D), jnp.float32),
        ],
        scratch_shapes=[
            pltpu.VMEM((NBLK, BLK), jnp.float32),
            pltpu.VMEM((NBLK, BLK), jnp.int32),
            pltpu.VMEM((NSTR, 128), jnp.float32),
            pltpu.VMEM((NSTR, 128), jnp.int32),
            pltpu.SMEM((NSTR, 128), jnp.float32),
            pltpu.SMEM((NSTR, 128), jnp.int32),
            pltpu.SemaphoreType.DMA,
        ],
        compiler_params=pltpu.CompilerParams(
            dimension_semantics=("arbitrary",)),
    )(embs, embs, w_pad)
    return scores[:KSEL], spans[:KSEL], cls[:KSEL], emb[:KSEL]


# speedup vs baseline: 1.8352x; 1.8352x over previous
import jax
import jax.numpy as jnp
from jax.experimental import pallas as pl
from jax.experimental.pallas import tpu as pltpu

N_ROWS = 32768
D = 768
NA = 18
NAPAD = 24
KSEL = 50
KPAD = 64
NBLK = 8
BLK = N_ROWS // NBLK
NSTR = 4
SW = BLK // NSTR


def _body(x_hbm, x_ref, w_ref, scores_out, spans_out, cls_out, emb_out,
          max_scr, key_scr, accv_scr, acck_scr, v_smem, k_smem, sem):
    g = pl.program_id(0)
    xb = x_ref[...].astype(jnp.bfloat16)
    st = jax.lax.dot_general(w_ref[...], xb, (((1,), (1,)), ((), ())),
                             preferred_element_type=jnp.float32)
    row = jax.lax.broadcasted_iota(jnp.int32, (NAPAD, 1), 0)
    m = jnp.max(st, axis=0)
    cls = jnp.min(jnp.where(st == m[None, :], row, NAPAD),
                  axis=0).astype(jnp.int32)
    col = jax.lax.iota(jnp.int32, BLK)
    max_scr[g, :] = m
    key_scr[g, :] = (g * BLK + col) * 32 + cls

    @pl.when(g == NBLK - 1)
    def _():
        lane = jax.lax.broadcasted_iota(jnp.int32, (1, 128), 1)
        BIGI = jnp.int32(2**30)
        a = max_scr[...]
        keys = key_scr[...]
        accv = jnp.zeros((1, 128), jnp.float32)
        acck = jnp.zeros((1, 128), jnp.int32)
        for i in range(KSEL):
            m1 = jnp.max(a, axis=1, keepdims=True)
            k1 = jnp.min(jnp.where(a == m1, keys, BIGI),
                         axis=1, keepdims=True)
            mv, kv = m1, k1
            for s in (4, 2, 1):
                mv2 = pltpu.roll(mv, s, 0)
                kv2 = pltpu.roll(kv, s, 0)
                t = (mv2 > mv) | ((mv2 == mv) & (kv2 < kv))
                mv = jnp.where(t, mv2, mv)
                kv = jnp.where(t, kv2, kv)
            eqj = keys == kv
            oh = lane == i
            accv = jnp.where(oh, jnp.broadcast_to(mv[0:1], (1, 128)), accv)
            acck = jnp.where(oh, jnp.broadcast_to(kv[0:1], (1, 128)), acck)
            a = jnp.where(eqj, -jnp.inf, a)
        accv_scr[0, :] = accv[0]
        acck_scr[0, :] = acck[0]
        cp = pltpu.make_async_copy(accv_scr, v_smem, sem)
        cp.start()
        cp.wait()
        cp = pltpu.make_async_copy(acck_scr, k_smem, sem)
        cp.start()
        cp.wait()
        for i in range(KSEL):
            scores_out[i] = v_smem[0, i]
            spans_out[i] = jax.lax.shift_right_logical(k_smem[0, i], 5)
            cls_out[i] = jax.lax.bitwise_and(k_smem[0, i], 31)
        for i in range(KSEL):
            pltpu.make_async_copy(
                x_hbm.at[pl.ds(spans_out[i], 1), :],
                emb_out.at[pl.ds(i, 1), :], sem).start()
        for i in range(KSEL):
            pltpu.make_async_copy(
                x_hbm.at[pl.ds(0, 1), :],
                emb_out.at[pl.ds(i, 1), :], sem).wait()


def kernel(embs, entity_anchor, k):
    del k
    w_pad = jnp.concatenate(
        [entity_anchor,
         jnp.broadcast_to(entity_anchor[:1], (NAPAD - NA, D))],
        axis=0).astype(jnp.bfloat16)
    scores, spans, cls, emb = pl.pallas_call(
        _body,
        grid=(NBLK,),
        in_specs=[
            pl.BlockSpec(memory_space=pl.ANY),
            pl.BlockSpec((BLK, D), lambda g: (g, 0)),
            pl.BlockSpec((NAPAD, D), lambda g: (0, 0)),
        ],
        out_specs=[
            pl.BlockSpec(memory_space=pltpu.SMEM),
            pl.BlockSpec(memory_space=pltpu.SMEM),
            pl.BlockSpec(memory_space=pltpu.SMEM),
            pl.BlockSpec((KPAD, D), lambda g: (0, 0)),
        ],
        out_shape=[
            jax.ShapeDtypeStruct((128,), jnp.float32),
            jax.ShapeDtypeStruct((128,), jnp.int32),
            jax.ShapeDtypeStruct((128,), jnp.int32),
            jax.ShapeDtypeStruct((KPAD, D), jnp.float32),
        ],
        scratch_shapes=[
            pltpu.VMEM((NBLK, BLK), jnp.float32),
            pltpu.VMEM((NBLK, BLK), jnp.int32),
            pltpu.VMEM((NSTR, 128), jnp.float32),
            pltpu.VMEM((NSTR, 128), jnp.int32),
            pltpu.SMEM((NSTR, 128), jnp.float32),
            pltpu.SMEM((NSTR, 128), jnp.int32),
            pltpu.SemaphoreType.DMA,
        ],
        compiler_params=pltpu.CompilerParams(
            dimension_semantics=("arbitrary",)),
    )(embs, embs, w_pad)
    return scores[:KSEL], spans[:KSEL], cls[:KSEL], emb[:KSEL]
